# Initial kernel scaffold; baseline (speedup 1.0000x reference)
import jax
import jax.numpy as jnp
from jax.experimental import pallas as pl


def kernel(x, norm1_w, q_w, q_b, k_w, k_b, v_w, v_b, o_w, o_b, sink_bias, norm2_w, router_w, router_b, W1, b1, W2, b2):
    raise NotImplementedError("write your pallas kernel here")


# speedup vs baseline: 1.2816x; 1.2816x over previous
import functools
import math

import jax
import jax.numpy as jnp
from jax import lax
from jax.experimental import pallas as pl
from jax.experimental.pallas import tpu as pltpu
from jax.experimental.pallas import tpu_sc as plsc

DIM = 768
H = 8
KVH = 2
HD = DIM // H
WIN = 64
E = 16
TOPK = 2
HID = 512
S = 2048
EPS = 1e-6
FP16_MIN = -65504.0

QBLK = 128
RBLK = 128
BLK = 64
PT = TOPK * S + E * BLK
NB = PT // BLK
NW = 32
TPW = S // NW
CHUNK = 32


def _qkv_body(x_ref, w_ref, b_ref, n1_ref, q_ref, k_ref, v_ref):
    x = x_ref[...]
    h = x * lax.rsqrt(jnp.mean(x * x, axis=-1, keepdims=True) + EPS) * n1_ref[...]
    qkv = jnp.dot(h, w_ref[...], preferred_element_type=jnp.float32) + b_ref[...]
    q_ref[...] = qkv[:, :DIM]
    k_ref[...] = qkv[:, DIM:DIM + KVH * HD]
    v_ref[...] = qkv[:, DIM + KVH * HD:]


def _qkv_call(x, wqkv, bqkv, n1):
    nblk = S // 256
    return pl.pallas_call(
        _qkv_body,
        grid=(nblk,),
        in_specs=[
            pl.BlockSpec((256, DIM), lambda i: (i, 0)),
            pl.BlockSpec((DIM, DIM + 2 * KVH * HD), lambda i: (0, 0)),
            pl.BlockSpec((1, DIM + 2 * KVH * HD), lambda i: (0, 0)),
            pl.BlockSpec((1, DIM), lambda i: (0, 0)),
        ],
        out_specs=[
            pl.BlockSpec((256, DIM), lambda i: (i, 0)),
            pl.BlockSpec((256, KVH * HD), lambda i: (i, 0)),
            pl.BlockSpec((256, KVH * HD), lambda i: (i, 0)),
        ],
        out_shape=[
            jax.ShapeDtypeStruct((S, DIM), jnp.float32),
            jax.ShapeDtypeStruct((S, KVH * HD), jnp.float32),
            jax.ShapeDtypeStruct((S, KVH * HD), jnp.float32),
        ],
    )(x, wqkv, bqkv, n1)


def _attn_body(sink_ref, q_ref, kp_ref, kc_ref, vp_ref, vc_ref, o_ref):
    i = pl.program_id(0)
    sink = sink_ref[0, 0]
    r = lax.broadcasted_iota(jnp.int32, (QBLK, 2 * QBLK), 0)
    c = lax.broadcasted_iota(jnp.int32, (QBLK, 2 * QBLK), 1)
    d = QBLK + r - c
    valid = (d >= 0) & (d < WIN)
    valid = valid & ((i >= 1) | (c >= QBLK))
    scale = 1.0 / math.sqrt(HD)
    for h in range(H):
        g = h // (H // KVH)
        qh = q_ref[:, HD * h:HD * (h + 1)]
        kh = jnp.concatenate(
            [kp_ref[:, HD * g:HD * (g + 1)], kc_ref[:, HD * g:HD * (g + 1)]], axis=0)
        vh = jnp.concatenate(
            [vp_ref[:, HD * g:HD * (g + 1)], vc_ref[:, HD * g:HD * (g + 1)]], axis=0)
        s = lax.dot_general(qh, kh, (((1,), (1,)), ((), ())),
                            preferred_element_type=jnp.float32) * scale
        s = jnp.where(valid, s, FP16_MIN)
        m = jnp.maximum(jnp.max(s, axis=-1, keepdims=True), sink)
        p = jnp.exp(s - m)
        denom = jnp.sum(p, axis=-1, keepdims=True) + jnp.exp(sink - m)
        o = jnp.dot(p, vh, preferred_element_type=jnp.float32) / denom
        o_ref[:, HD * h:HD * (h + 1)] = o


def _attn_call(sink, q, k, v):
    nblk = S // QBLK
    prev_map = lambda i: (jnp.maximum(i - 1, 0), 0)
    cur_map = lambda i: (i, 0)
    return pl.pallas_call(
        _attn_body,
        grid=(nblk,),
        in_specs=[
            pl.BlockSpec((1, 1), lambda i: (0, 0)),
            pl.BlockSpec((QBLK, DIM), cur_map),
            pl.BlockSpec((QBLK, KVH * HD), prev_map),
            pl.BlockSpec((QBLK, KVH * HD), cur_map),
            pl.BlockSpec((QBLK, KVH * HD), prev_map),
            pl.BlockSpec((QBLK, KVH * HD), cur_map),
        ],
        out_specs=pl.BlockSpec((QBLK, DIM), cur_map),
        out_shape=jax.ShapeDtypeStruct((S, DIM), jnp.float32),
    )(sink, q, k, k, v, v)


def _post_body(x_ref, a_ref, ow_ref, ob_ref, n2_ref, rw_ref, rb_ref, lt_ref,
               x2_ref, h2_ref, e1_ref, e2_ref, r1_ref, r2_ref, w1_ref, w2_ref,
               g_ref, rps_ref):
    i = pl.program_id(0)

    @pl.when(i == 0)
    def _():
        g_ref[...] = jnp.zeros_like(g_ref)
        rps_ref[...] = jnp.zeros_like(rps_ref)

    x2 = (jnp.dot(a_ref[...], ow_ref[...], preferred_element_type=jnp.float32)
          + ob_ref[...] + x_ref[...])
    x2_ref[...] = x2
    h2 = x2 * lax.rsqrt(jnp.mean(x2 * x2, axis=-1, keepdims=True) + EPS) * n2_ref[...]
    h2_ref[...] = h2
    logits = (jnp.dot(h2, rw_ref[...], preferred_element_type=jnp.float32)
              + rb_ref[...]) * 10.0
    idx = lax.broadcasted_iota(jnp.int32, (RBLK, E), 1)
    m1 = jnp.max(logits, axis=-1, keepdims=True)
    i1 = jnp.min(jnp.where(logits == m1, idx, E + 1), axis=-1, keepdims=True)
    l2 = jnp.where(idx == i1, -jnp.inf, logits)
    m2 = jnp.max(l2, axis=-1, keepdims=True)
    i2 = jnp.min(jnp.where(l2 == m2, idx, E + 1), axis=-1, keepdims=True)
    b = jnp.exp(m2 - m1)
    w1 = 1.0 / (1.0 + b)
    w2 = b * w1
    p = jnp.exp(logits - m1)
    rp = p / jnp.sum(p, axis=-1, keepdims=True)
    rps_ref[0, 0, :] += jnp.sum(rp, axis=0)
    hot = (idx == i1).astype(jnp.float32) + (idx == i2).astype(jnp.float32)
    prev = g_ref[0, 0, :]
    cnt = prev[None, :] + jnp.dot(lt_ref[...], hot, preferred_element_type=jnp.float32)
    g_ref[0, 0, :] = prev + jnp.sum(hot, axis=0)
    r1 = jnp.sum(jnp.where(idx == i1, cnt, 0.0), axis=-1)
    r2 = jnp.sum(jnp.where(idx == i2, cnt, 0.0), axis=-1)
    e1_ref[...] = i1[:, 0].reshape(1, 1, RBLK)
    e2_ref[...] = i2[:, 0].reshape(1, 1, RBLK)
    r1_ref[...] = r1.astype(jnp.int32).reshape(1, 1, RBLK)
    r2_ref[...] = r2.astype(jnp.int32).reshape(1, 1, RBLK)
    w1_ref[...] = w1[:, 0].reshape(1, 1, RBLK)
    w2_ref[...] = w2[:, 0].reshape(1, 1, RBLK)


def _post_call(x, attn, ow, ob, n2, rw, rb, lt):
    nblk = S // RBLK
    small = lambda dt: jax.ShapeDtypeStruct((nblk, 1, RBLK), dt)
    small_spec = pl.BlockSpec((1, 1, RBLK), lambda i: (i, 0, 0))
    return pl.pallas_call(
        _post_body,
        grid=(nblk,),
        in_specs=[
            pl.BlockSpec((RBLK, DIM), lambda i: (i, 0)),
            pl.BlockSpec((RBLK, DIM), lambda i: (i, 0)),
            pl.BlockSpec((DIM, DIM), lambda i: (0, 0)),
            pl.BlockSpec((1, DIM), lambda i: (0, 0)),
            pl.BlockSpec((1, DIM), lambda i: (0, 0)),
            pl.BlockSpec((DIM, E), lambda i: (0, 0)),
            pl.BlockSpec((1, E), lambda i: (0, 0)),
            pl.BlockSpec((RBLK, RBLK), lambda i: (0, 0)),
        ],
        out_specs=[
            pl.BlockSpec((RBLK, DIM), lambda i: (i, 0)),
            pl.BlockSpec((RBLK, DIM), lambda i: (i, 0)),
            small_spec, small_spec, small_spec, small_spec, small_spec, small_spec,
            pl.BlockSpec((1, 1, E), lambda i: (0, 0, 0)),
            pl.BlockSpec((1, 1, E), lambda i: (0, 0, 0)),
        ],
        out_shape=[
            jax.ShapeDtypeStruct((S, DIM), jnp.float32),
            jax.ShapeDtypeStruct((S, DIM), jnp.float32),
            small(jnp.int32), small(jnp.int32), small(jnp.int32), small(jnp.int32),
            small(jnp.float32), small(jnp.float32),
            jax.ShapeDtypeStruct((1, 1, E), jnp.float32),
            jax.ShapeDtypeStruct((1, 1, E), jnp.float32),
        ],
    )(x, attn, ow, ob, n2, rw, rb, lt)


def _sc_dispatch_call(h2, e1, r1, e2, r2, g):
    mesh = plsc.VectorSubcoreMesh(core_axis_name="c", subcore_axis_name="s")

    @functools.partial(
        pl.kernel, mesh=mesh,
        compiler_params=pltpu.CompilerParams(needs_layout_passes=False),
        out_type=[
            jax.ShapeDtypeStruct((PT, DIM), jnp.float32),
            jax.ShapeDtypeStruct((S,), jnp.int32),
            jax.ShapeDtypeStruct((S,), jnp.int32),
            jax.ShapeDtypeStruct((NB,), jnp.int32),
        ],
        scratch_types=[
            pltpu.VMEM((E,), jnp.int32),
            pltpu.VMEM((E,), jnp.int32),
            pltpu.VMEM((TPW,), jnp.int32),
            pltpu.VMEM((TPW,), jnp.int32),
            pltpu.VMEM((TPW,), jnp.int32),
            pltpu.VMEM((TPW,), jnp.int32),
            pltpu.VMEM((TPW,), jnp.int32),
            pltpu.VMEM((TPW,), jnp.int32),
            pltpu.VMEM((TPW, DIM), jnp.float32),
            pltpu.VMEM((NB,), jnp.int32),
            pltpu.SemaphoreType.DMA,
            pltpu.SemaphoreType.DMA,
        ],
    )
    def disp(h2_hbm, e1_hbm, r1_hbm, e2_hbm, r2_hbm, g_hbm,
             a_hbm, p1_hbm, p2_hbm, be_hbm,
             g_v, po_v, e1_v, r1_v, e2_v, r2_v, p1_v, p2_v, rows_v, be_v,
             sem1, sem2):
        wid = lax.axis_index("s") * 2 + lax.axis_index("c")
        base = wid * TPW
        pltpu.sync_copy(g_hbm, g_v)
        gv = g_v[...]
        rounded = ((gv + (BLK - 1)) >> 6) << 6
        lane = lax.iota(jnp.int32, 16)
        po_v[...] = rounded
        for sft in (1, 2, 4, 8):
            cur = po_v[...]
            shifted = plsc.load_gather(po_v, [jnp.maximum(lane - sft, 0)])
            po_v[...] = cur + jnp.where(lane >= sft, shifted, 0)
        po_v[...] = po_v[...] - rounded
        pltpu.sync_copy(e1_hbm.at[pl.ds(base, TPW)], e1_v)
        pltpu.sync_copy(r1_hbm.at[pl.ds(base, TPW)], r1_v)
        pltpu.sync_copy(e2_hbm.at[pl.ds(base, TPW)], e2_v)
        pltpu.sync_copy(r2_hbm.at[pl.ds(base, TPW)], r2_v)
        for j in range(TPW // 16):
            sl = pl.ds(j * 16, 16)
            p1_v[sl] = plsc.load_gather(po_v, [e1_v[sl]]) + r1_v[sl]
            p2_v[sl] = plsc.load_gather(po_v, [e2_v[sl]]) + r2_v[sl]
        pltpu.sync_copy(p1_v, p1_hbm.at[pl.ds(base, TPW)])
        pltpu.sync_copy(p2_v, p2_hbm.at[pl.ds(base, TPW)])
        pltpu.sync_copy(h2_hbm.at[pl.ds(base, TPW)], rows_v)
        cp1 = pltpu.async_copy(rows_v, a_hbm.at[p1_v], sem1)
        cp2 = pltpu.async_copy(rows_v, a_hbm.at[p2_v], sem2)

        @pl.when(wid == 0)
        def _():
            for bv in range(NB // 16):
                bidx = (lax.iota(jnp.int32, 16) + bv * 16) * BLK
                acc = jnp.zeros((16,), jnp.int32)
                for e in range(E):
                    poe = plsc.load_gather(po_v, [jnp.full((16,), e, jnp.int32)])
                    acc += jnp.where(bidx >= poe, 1, 0)
                be_v[pl.ds(bv * 16, 16)] = acc - 1
            pltpu.sync_copy(be_v, be_hbm)

        cp1.wait()
        cp2.wait()

    return disp(h2, e1, r1, e2, r2, g)


def _ffn_body(be_ref, a_ref, w1_ref, b1_ref, w2_ref, b2_ref, y_ref):
    a = a_ref[...]
    hm = jnp.dot(a, w1_ref[0], preferred_element_type=jnp.float32) + b1_ref[0]
    hm = hm * jax.nn.sigmoid(hm)
    y_ref[...] = jnp.dot(hm, w2_ref[0], preferred_element_type=jnp.float32) + b2_ref[0]


def _ffn_call(be, a, w1, b1, w2, b2):
    grid_spec = pltpu.PrefetchScalarGridSpec(
        num_scalar_prefetch=1,
        grid=(NB,),
        in_specs=[
            pl.BlockSpec((BLK, DIM), lambda b, be: (b, 0)),
            pl.BlockSpec((1, DIM, HID), lambda b, be: (be[b], 0, 0)),
            pl.BlockSpec((1, 1, HID), lambda b, be: (be[b], 0, 0)),
            pl.BlockSpec((1, HID, DIM), lambda b, be: (be[b], 0, 0)),
            pl.BlockSpec((1, 1, DIM), lambda b, be: (be[b], 0, 0)),
        ],
        out_specs=pl.BlockSpec((BLK, DIM), lambda b, be: (b, 0)),
    )
    return pl.pallas_call(
        _ffn_body,
        grid_spec=grid_spec,
        out_shape=jax.ShapeDtypeStruct((PT, DIM), jnp.float32),
    )(be, a, w1, b1.reshape(E, 1, HID), w2, b2.reshape(E, 1, DIM))


def _sc_combine_call(x2, y, p1, p2, w1, w2):
    mesh = plsc.VectorSubcoreMesh(core_axis_name="c", subcore_axis_name="s")

    @functools.partial(
        pl.kernel, mesh=mesh,
        compiler_params=pltpu.CompilerParams(needs_layout_passes=False),
        out_type=jax.ShapeDtypeStruct((S, DIM), jnp.float32),
        scratch_types=[
            pltpu.VMEM((CHUNK,), jnp.int32),
            pltpu.VMEM((CHUNK,), jnp.int32),
            pltpu.VMEM((CHUNK,), jnp.float32),
            pltpu.VMEM((CHUNK,), jnp.float32),
            pltpu.VMEM((CHUNK, DIM), jnp.float32),
            pltpu.VMEM((CHUNK, DIM), jnp.float32),
            pltpu.VMEM((CHUNK, DIM), jnp.float32),
            pltpu.SemaphoreType.DMA,
            pltpu.SemaphoreType.DMA,
        ],
    )
    def comb(x2_hbm, y_hbm, p1_hbm, p2_hbm, w1_hbm, w2_hbm, out_hbm,
             p1_v, p2_v, w1_v, w2_v, y1_v, y2_v, xr_v, sem1, sem2):
        wid = lax.axis_index("s") * 2 + lax.axis_index("c")
        for ci in range(TPW // CHUNK):
            cbase = wid * TPW + ci * CHUNK
            pltpu.sync_copy(p1_hbm.at[pl.ds(cbase, CHUNK)], p1_v)
            pltpu.sync_copy(p2_hbm.at[pl.ds(cbase, CHUNK)], p2_v)
            pltpu.sync_copy(w1_hbm.at[pl.ds(cbase, CHUNK)], w1_v)
            pltpu.sync_copy(w2_hbm.at[pl.ds(cbase, CHUNK)], w2_v)
            cp1 = pltpu.async_copy(y_hbm.at[p1_v], y1_v, sem1)
            cp2 = pltpu.async_copy(y_hbm.at[p2_v], y2_v, sem2)
            pltpu.sync_copy(x2_hbm.at[pl.ds(cbase, CHUNK)], xr_v)
            cp1.wait()
            cp2.wait()

            def row_fn(i, _):
                s1 = plsc.load_gather(w1_v, [jnp.full((16,), i, jnp.int32)])
                s2 = plsc.load_gather(w2_v, [jnp.full((16,), i, jnp.int32)])
                for cc in range(DIM // 16):
                    sl = pl.ds(cc * 16, 16)
                    xr_v[i, sl] = (xr_v[i, sl] + y1_v[i, sl] * s1
                                   + y2_v[i, sl] * s2)
                return 0

            lax.fori_loop(0, CHUNK, row_fn, 0)
            pltpu.sync_copy(xr_v, out_hbm.at[pl.ds(cbase, CHUNK)])

    return comb(x2, y, p1, p2, w1, w2)


def kernel(x, norm1_w, q_w, q_b, k_w, k_b, v_w, v_b, o_w, o_b, sink_bias,
           norm2_w, router_w, router_b, W1, b1, W2, b2):
    xf = x.reshape(S, DIM)
    wqkv = jnp.concatenate([q_w, k_w, v_w], axis=1)
    bqkv = jnp.concatenate([q_b, k_b, v_b]).reshape(1, -1)
    q, k, v = _qkv_call(xf, wqkv, bqkv, norm1_w.reshape(1, DIM))
    attn = _attn_call(jnp.reshape(sink_bias, (1, 1)), q, k, v)
    lt = jnp.tril(jnp.ones((RBLK, RBLK), jnp.float32), -1)
    (x2, h2, e1, e2, r1, r2, w1t, w2t, g, rps) = _post_call(
        xf, attn, o_w, o_b.reshape(1, DIM), norm2_w.reshape(1, DIM),
        router_w, router_b.reshape(1, E), lt)
    e1 = e1.reshape(S)
    e2 = e2.reshape(S)
    r1 = r1.reshape(S)
    r2 = r2.reshape(S)
    w1t = w1t.reshape(S)
    w2t = w2t.reshape(S)
    gi = g.reshape(E).astype(jnp.int32)
    a, p1, p2, be = _sc_dispatch_call(h2, e1, r1, e2, r2, gi)
    y = _ffn_call(be, a, W1, b1, W2, b2)
    out = _sc_combine_call(x2, y, p1, p2, w1t, w2t)
    rpsf = rps.reshape(E)
    aux = jnp.sum(rpsf * rpsf) / E * 1e-05
    return out.reshape(1, S, DIM), aux


# speedup vs baseline: 1.3519x; 1.0548x over previous
import functools
import math

import jax
import jax.numpy as jnp
from jax import lax
from jax.experimental import pallas as pl
from jax.experimental.pallas import tpu as pltpu
from jax.experimental.pallas import tpu_sc as plsc

DIM = 768
H = 8
KVH = 2
HD = DIM // H
WIN = 64
E = 16
TOPK = 2
HID = 512
S = 2048
EPS = 1e-6
FP16_MIN = -65504.0

QBLK = 256
RBLK = 256
BLK = 64
PT = TOPK * S + E * BLK
NB = PT // BLK
NW = 32
TPW = S // NW
CHUNK = 32


def _qkv_body(x_ref, w_ref, b_ref, n1_ref, q_ref, k_ref, v_ref):
    x = x_ref[...]
    h = x * lax.rsqrt(jnp.mean(x * x, axis=-1, keepdims=True) + EPS) * n1_ref[...]
    qkv = jnp.dot(h.astype(jnp.bfloat16), w_ref[...],
                  preferred_element_type=jnp.float32) + b_ref[...]
    q_ref[...] = qkv[:, :DIM].astype(jnp.bfloat16)
    k_ref[...] = qkv[:, DIM:DIM + KVH * HD].astype(jnp.bfloat16)
    v_ref[...] = qkv[:, DIM + KVH * HD:].astype(jnp.bfloat16)


def _qkv_call(x, wqkv, bqkv, n1):
    nblk = S // 256
    return pl.pallas_call(
        _qkv_body,
        grid=(nblk,),
        in_specs=[
            pl.BlockSpec((256, DIM), lambda i: (i, 0)),
            pl.BlockSpec((DIM, DIM + 2 * KVH * HD), lambda i: (0, 0)),
            pl.BlockSpec((1, DIM + 2 * KVH * HD), lambda i: (0, 0)),
            pl.BlockSpec((1, DIM), lambda i: (0, 0)),
        ],
        out_specs=[
            pl.BlockSpec((256, DIM), lambda i: (i, 0)),
            pl.BlockSpec((256, KVH * HD), lambda i: (i, 0)),
            pl.BlockSpec((256, KVH * HD), lambda i: (i, 0)),
        ],
        out_shape=[
            jax.ShapeDtypeStruct((S, DIM), jnp.bfloat16),
            jax.ShapeDtypeStruct((S, KVH * HD), jnp.bfloat16),
            jax.ShapeDtypeStruct((S, KVH * HD), jnp.bfloat16),
        ],
    )(x, wqkv, bqkv, n1)


def _attn_body(sink_ref, q_ref, kp_ref, kc_ref, vp_ref, vc_ref, o_ref):
    i = pl.program_id(0)
    sink = sink_ref[0, 0]
    r = lax.broadcasted_iota(jnp.int32, (QBLK, 2 * QBLK), 0)
    c = lax.broadcasted_iota(jnp.int32, (QBLK, 2 * QBLK), 1)
    d = QBLK + r - c
    valid = (d >= 0) & (d < WIN)
    valid = valid & ((i >= 1) | (c >= QBLK))
    scale = 1.0 / math.sqrt(HD)
    for h in range(H):
        g = h // (H // KVH)
        qh = q_ref[:, HD * h:HD * (h + 1)]
        kh = jnp.concatenate(
            [kp_ref[:, HD * g:HD * (g + 1)], kc_ref[:, HD * g:HD * (g + 1)]], axis=0)
        vh = jnp.concatenate(
            [vp_ref[:, HD * g:HD * (g + 1)], vc_ref[:, HD * g:HD * (g + 1)]], axis=0)
        s = lax.dot_general(qh, kh, (((1,), (1,)), ((), ())),
                            preferred_element_type=jnp.float32) * scale
        s = jnp.where(valid, s, FP16_MIN)
        m = jnp.maximum(jnp.max(s, axis=-1, keepdims=True), sink)
        p = jnp.exp(s - m)
        denom = jnp.sum(p, axis=-1, keepdims=True) + jnp.exp(sink - m)
        o = jnp.dot(p.astype(jnp.bfloat16), vh,
                    preferred_element_type=jnp.float32) / denom
        o_ref[:, HD * h:HD * (h + 1)] = o.astype(jnp.bfloat16)


def _attn_call(sink, q, k, v):
    nblk = S // QBLK
    prev_map = lambda i: (jnp.maximum(i - 1, 0), 0)
    cur_map = lambda i: (i, 0)
    return pl.pallas_call(
        _attn_body,
        grid=(nblk,),
        in_specs=[
            pl.BlockSpec((1, 1), lambda i: (0, 0)),
            pl.BlockSpec((QBLK, DIM), cur_map),
            pl.BlockSpec((QBLK, KVH * HD), prev_map),
            pl.BlockSpec((QBLK, KVH * HD), cur_map),
            pl.BlockSpec((QBLK, KVH * HD), prev_map),
            pl.BlockSpec((QBLK, KVH * HD), cur_map),
        ],
        out_specs=pl.BlockSpec((QBLK, DIM), cur_map),
        out_shape=jax.ShapeDtypeStruct((S, DIM), jnp.bfloat16),
    )(sink, q, k, k, v, v)


def _post_body(x_ref, a_ref, ow_ref, ob_ref, n2_ref, rw_ref, rb_ref,
               x2_ref, h2_ref, e1_ref, e2_ref, r1_ref, r2_ref, w1_ref, w2_ref,
               g_ref, rps_ref):
    i = pl.program_id(0)

    @pl.when(i == 0)
    def _():
        g_ref[...] = jnp.zeros_like(g_ref)
        rps_ref[...] = jnp.zeros_like(rps_ref)

    x2 = (jnp.dot(a_ref[...], ow_ref[...], preferred_element_type=jnp.float32)
          + ob_ref[...] + x_ref[...])
    x2_ref[...] = x2
    h2 = x2 * lax.rsqrt(jnp.mean(x2 * x2, axis=-1, keepdims=True) + EPS) * n2_ref[...]
    h2_ref[...] = h2
    logits = (jnp.dot(h2, rw_ref[...], preferred_element_type=jnp.float32)
              + rb_ref[...]) * 10.0
    idx = lax.broadcasted_iota(jnp.int32, (RBLK, E), 1)
    m1 = jnp.max(logits, axis=-1, keepdims=True)
    i1 = jnp.min(jnp.where(logits == m1, idx, E + 1), axis=-1, keepdims=True)
    l2 = jnp.where(idx == i1, -jnp.inf, logits)
    m2 = jnp.max(l2, axis=-1, keepdims=True)
    i2 = jnp.min(jnp.where(l2 == m2, idx, E + 1), axis=-1, keepdims=True)
    b = jnp.exp(m2 - m1)
    w1 = 1.0 / (1.0 + b)
    w2 = b * w1
    p = jnp.exp(logits - m1)
    rp = p / jnp.sum(p, axis=-1, keepdims=True)
    rps_ref[0, 0, :] += jnp.sum(rp, axis=0)
    hot = (idx == i1).astype(jnp.float32) + (idx == i2).astype(jnp.float32)
    ri = lax.broadcasted_iota(jnp.int32, (RBLK, RBLK), 0)
    ci = lax.broadcasted_iota(jnp.int32, (RBLK, RBLK), 1)
    lt = (ci < ri).astype(jnp.float32)
    prev = g_ref[0, 0, :]
    cnt = prev[None, :] + jnp.dot(lt, hot, preferred_element_type=jnp.float32)
    g_ref[0, 0, :] = prev + jnp.sum(hot, axis=0)
    r1 = jnp.sum(jnp.where(idx == i1, cnt, 0.0), axis=-1)
    r2 = jnp.sum(jnp.where(idx == i2, cnt, 0.0), axis=-1)
    e1_ref[...] = i1[:, 0].reshape(1, 1, RBLK)
    e2_ref[...] = i2[:, 0].reshape(1, 1, RBLK)
    r1_ref[...] = r1.astype(jnp.int32).reshape(1, 1, RBLK)
    r2_ref[...] = r2.astype(jnp.int32).reshape(1, 1, RBLK)
    w1_ref[...] = w1[:, 0].reshape(1, 1, RBLK)
    w2_ref[...] = w2[:, 0].reshape(1, 1, RBLK)


def _post_call(x, attn, ow, ob, n2, rw, rb):
    nblk = S // RBLK
    small = lambda dt: jax.ShapeDtypeStruct((nblk, 1, RBLK), dt)
    small_spec = pl.BlockSpec((1, 1, RBLK), lambda i: (i, 0, 0))
    return pl.pallas_call(
        _post_body,
        grid=(nblk,),
        in_specs=[
            pl.BlockSpec((RBLK, DIM), lambda i: (i, 0)),
            pl.BlockSpec((RBLK, DIM), lambda i: (i, 0)),
            pl.BlockSpec((DIM, DIM), lambda i: (0, 0)),
            pl.BlockSpec((1, DIM), lambda i: (0, 0)),
            pl.BlockSpec((1, DIM), lambda i: (0, 0)),
            pl.BlockSpec((DIM, E), lambda i: (0, 0)),
            pl.BlockSpec((1, E), lambda i: (0, 0)),
        ],
        out_specs=[
            pl.BlockSpec((RBLK, DIM), lambda i: (i, 0)),
            pl.BlockSpec((RBLK, DIM), lambda i: (i, 0)),
            small_spec, small_spec, small_spec, small_spec, small_spec, small_spec,
            pl.BlockSpec((1, 1, E), lambda i: (0, 0, 0)),
            pl.BlockSpec((1, 1, E), lambda i: (0, 0, 0)),
        ],
        out_shape=[
            jax.ShapeDtypeStruct((S, DIM), jnp.float32),
            jax.ShapeDtypeStruct((S, DIM), jnp.float32),
            small(jnp.int32), small(jnp.int32), small(jnp.int32), small(jnp.int32),
            small(jnp.float32), small(jnp.float32),
            jax.ShapeDtypeStruct((1, 1, E), jnp.float32),
            jax.ShapeDtypeStruct((1, 1, E), jnp.float32),
        ],
    )(x, attn, ow, ob, n2, rw, rb)


def _sc_dispatch_call(h2, e1, r1, e2, r2, g):
    mesh = plsc.VectorSubcoreMesh(core_axis_name="c", subcore_axis_name="s")

    @functools.partial(
        pl.kernel, mesh=mesh,
        compiler_params=pltpu.CompilerParams(needs_layout_passes=False),
        out_type=[
            jax.ShapeDtypeStruct((PT, DIM), jnp.float32),
            jax.ShapeDtypeStruct((S,), jnp.int32),
            jax.ShapeDtypeStruct((S,), jnp.int32),
            jax.ShapeDtypeStruct((NB,), jnp.int32),
        ],
        scratch_types=[
            pltpu.VMEM((E,), jnp.int32),
            pltpu.VMEM((E,), jnp.int32),
            pltpu.VMEM((TPW,), jnp.int32),
            pltpu.VMEM((TPW,), jnp.int32),
            pltpu.VMEM((TPW,), jnp.int32),
            pltpu.VMEM((TPW,), jnp.int32),
            pltpu.VMEM((TPW,), jnp.int32),
            pltpu.VMEM((TPW,), jnp.int32),
            pltpu.VMEM((TPW, DIM), jnp.float32),
            pltpu.VMEM((NB,), jnp.int32),
            pltpu.SemaphoreType.DMA,
            pltpu.SemaphoreType.DMA,
        ],
    )
    def disp(h2_hbm, e1_hbm, r1_hbm, e2_hbm, r2_hbm, g_hbm,
             a_hbm, p1_hbm, p2_hbm, be_hbm,
             g_v, po_v, e1_v, r1_v, e2_v, r2_v, p1_v, p2_v, rows_v, be_v,
             sem1, sem2):
        wid = lax.axis_index("s") * 2 + lax.axis_index("c")
        base = wid * TPW
        pltpu.sync_copy(g_hbm, g_v)
        gv = g_v[...]
        rounded = ((gv + (BLK - 1)) >> 6) << 6
        lane = lax.iota(jnp.int32, 16)
        po_v[...] = rounded
        for sft in (1, 2, 4, 8):
            cur = po_v[...]
            shifted = plsc.load_gather(po_v, [jnp.maximum(lane - sft, 0)])
            po_v[...] = cur + jnp.where(lane >= sft, shifted, 0)
        po_v[...] = po_v[...] - rounded
        pltpu.sync_copy(e1_hbm.at[pl.ds(base, TPW)], e1_v)
        pltpu.sync_copy(r1_hbm.at[pl.ds(base, TPW)], r1_v)
        pltpu.sync_copy(e2_hbm.at[pl.ds(base, TPW)], e2_v)
        pltpu.sync_copy(r2_hbm.at[pl.ds(base, TPW)], r2_v)
        for j in range(TPW // 16):
            sl = pl.ds(j * 16, 16)
            p1_v[sl] = plsc.load_gather(po_v, [e1_v[sl]]) + r1_v[sl]
            p2_v[sl] = plsc.load_gather(po_v, [e2_v[sl]]) + r2_v[sl]
        pltpu.sync_copy(p1_v, p1_hbm.at[pl.ds(base, TPW)])
        pltpu.sync_copy(p2_v, p2_hbm.at[pl.ds(base, TPW)])
        pltpu.sync_copy(h2_hbm.at[pl.ds(base, TPW)], rows_v)
        cp1 = pltpu.async_copy(rows_v, a_hbm.at[p1_v], sem1)
        cp2 = pltpu.async_copy(rows_v, a_hbm.at[p2_v], sem2)

        @pl.when(wid == 0)
        def _():
            for bv in range(NB // 16):
                bidx = (lax.iota(jnp.int32, 16) + bv * 16) * BLK
                acc = jnp.zeros((16,), jnp.int32)
                for e in range(E):
                    poe = plsc.load_gather(po_v, [jnp.full((16,), e, jnp.int32)])
                    acc += jnp.where(bidx >= poe, 1, 0)
                be_v[pl.ds(bv * 16, 16)] = acc - 1
            pltpu.sync_copy(be_v, be_hbm)

        cp1.wait()
        cp2.wait()

    return disp(h2, e1, r1, e2, r2, g)


def _ffn_body(be_ref, a_ref, w1_ref, b1_ref, w2_ref, b2_ref, y_ref,
              w1c_ref, w2c_ref):
    b = pl.program_id(0)
    changed = (b == 0) | (be_ref[b] != be_ref[jnp.maximum(b - 1, 0)])

    @pl.when(changed)
    def _():
        w1c_ref[...] = w1_ref[0].astype(jnp.bfloat16)
        w2c_ref[...] = w2_ref[0].astype(jnp.bfloat16)

    a = a_ref[...].astype(jnp.bfloat16)
    hm = jnp.dot(a, w1c_ref[...], preferred_element_type=jnp.float32) + b1_ref[0]
    hm = hm * jax.nn.sigmoid(hm)
    y_ref[...] = jnp.dot(hm.astype(jnp.bfloat16), w2c_ref[...],
                         preferred_element_type=jnp.float32) + b2_ref[0]


def _ffn_call(be, a, w1, b1, w2, b2):
    grid_spec = pltpu.PrefetchScalarGridSpec(
        num_scalar_prefetch=1,
        grid=(NB,),
        in_specs=[
            pl.BlockSpec((BLK, DIM), lambda b, be: (b, 0)),
            pl.BlockSpec((1, DIM, HID), lambda b, be: (be[b], 0, 0)),
            pl.BlockSpec((1, 1, HID), lambda b, be: (be[b], 0, 0)),
            pl.BlockSpec((1, HID, DIM), lambda b, be: (be[b], 0, 0)),
            pl.BlockSpec((1, 1, DIM), lambda b, be: (be[b], 0, 0)),
        ],
        out_specs=pl.BlockSpec((BLK, DIM), lambda b, be: (b, 0)),
        scratch_shapes=[
            pltpu.VMEM((DIM, HID), jnp.bfloat16),
            pltpu.VMEM((HID, DIM), jnp.bfloat16),
        ],
    )
    return pl.pallas_call(
        _ffn_body,
        grid_spec=grid_spec,
        out_shape=jax.ShapeDtypeStruct((PT, DIM), jnp.float32),
    )(be, a, w1, b1.reshape(E, 1, HID), w2, b2.reshape(E, 1, DIM))


def _sc_combine_call(x2, y, p1, p2, w1, w2):
    mesh = plsc.VectorSubcoreMesh(core_axis_name="c", subcore_axis_name="s")

    @functools.partial(
        pl.kernel, mesh=mesh,
        compiler_params=pltpu.CompilerParams(needs_layout_passes=False),
        out_type=jax.ShapeDtypeStruct((S, DIM), jnp.float32),
        scratch_types=[
            pltpu.VMEM((CHUNK,), jnp.int32),
            pltpu.VMEM((CHUNK,), jnp.int32),
            pltpu.VMEM((CHUNK,), jnp.float32),
            pltpu.VMEM((CHUNK,), jnp.float32),
            pltpu.VMEM((CHUNK, DIM), jnp.float32),
            pltpu.VMEM((CHUNK, DIM), jnp.float32),
            pltpu.VMEM((CHUNK, DIM), jnp.float32),
            pltpu.SemaphoreType.DMA,
            pltpu.SemaphoreType.DMA,
        ],
    )
    def comb(x2_hbm, y_hbm, p1_hbm, p2_hbm, w1_hbm, w2_hbm, out_hbm,
             p1_v, p2_v, w1_v, w2_v, y1_v, y2_v, xr_v, sem1, sem2):
        wid = lax.axis_index("s") * 2 + lax.axis_index("c")
        for ci in range(TPW // CHUNK):
            cbase = wid * TPW + ci * CHUNK
            pltpu.sync_copy(p1_hbm.at[pl.ds(cbase, CHUNK)], p1_v)
            pltpu.sync_copy(p2_hbm.at[pl.ds(cbase, CHUNK)], p2_v)
            pltpu.sync_copy(w1_hbm.at[pl.ds(cbase, CHUNK)], w1_v)
            pltpu.sync_copy(w2_hbm.at[pl.ds(cbase, CHUNK)], w2_v)
            cp1 = pltpu.async_copy(y_hbm.at[p1_v], y1_v, sem1)
            cp2 = pltpu.async_copy(y_hbm.at[p2_v], y2_v, sem2)
            pltpu.sync_copy(x2_hbm.at[pl.ds(cbase, CHUNK)], xr_v)
            cp1.wait()
            cp2.wait()

            def row_fn(i, _):
                s1 = plsc.load_gather(w1_v, [jnp.full((16,), i, jnp.int32)])
                s2 = plsc.load_gather(w2_v, [jnp.full((16,), i, jnp.int32)])
                for cc in range(DIM // 16):
                    sl = pl.ds(cc * 16, 16)
                    xr_v[i, sl] = (xr_v[i, sl] + y1_v[i, sl] * s1
                                   + y2_v[i, sl] * s2)
                return 0

            lax.fori_loop(0, CHUNK, row_fn, 0)
            pltpu.sync_copy(xr_v, out_hbm.at[pl.ds(cbase, CHUNK)])

    return comb(x2, y, p1, p2, w1, w2)


def kernel(x, norm1_w, q_w, q_b, k_w, k_b, v_w, v_b, o_w, o_b, sink_bias,
           norm2_w, router_w, router_b, W1, b1, W2, b2):
    xf = x.reshape(S, DIM)
    wqkv = jnp.concatenate([q_w, k_w, v_w], axis=1).astype(jnp.bfloat16)
    bqkv = jnp.concatenate([q_b, k_b, v_b]).reshape(1, -1)
    q, k, v = _qkv_call(xf, wqkv, bqkv, norm1_w.reshape(1, DIM))
    attn = _attn_call(jnp.reshape(sink_bias, (1, 1)), q, k, v)
    (x2, h2, e1, e2, r1, r2, w1t, w2t, g, rps) = _post_call(
        xf, attn, o_w.astype(jnp.bfloat16), o_b.reshape(1, DIM),
        norm2_w.reshape(1, DIM), router_w, router_b.reshape(1, E))
    e1 = e1.reshape(S)
    e2 = e2.reshape(S)
    r1 = r1.reshape(S)
    r2 = r2.reshape(S)
    w1t = w1t.reshape(S)
    w2t = w2t.reshape(S)
    gi = g.reshape(E).astype(jnp.int32)
    a, p1, p2, be = _sc_dispatch_call(h2, e1, r1, e2, r2, gi)
    y = _ffn_call(be, a, W1, b1, W2, b2)
    out = _sc_combine_call(x2, y, p1, p2, w1t, w2t)
    rpsf = rps.reshape(E)
    aux = jnp.sum(rpsf * rpsf) / E * 1e-05
    return out.reshape(1, S, DIM), aux
